# Initial kernel scaffold; baseline (speedup 1.0000x reference)
import jax
import jax.numpy as jnp
from jax.experimental import pallas as pl


def kernel(x):
    raise NotImplementedError("write your pallas kernel here")


# speedup vs baseline: 24.4851x; 24.4851x over previous
import jax
import jax.numpy as jnp
from jax import lax
from jax.experimental import pallas as pl
from jax.experimental.pallas import tpu as pltpu

_MAX_OBJ = 100
_IOU_THR = 0.45
_SCORE_THR = 0.25
_BIG_I32 = 2**30


def _nms_tc_kernel(x_ref,
                   nd_ref, sc_ref, cl_ref, ix_ref,
                   b0_ref, b1_ref, b2_ref, b3_ref,
                   s_scr, cls_scr, x1_scr, y1_scr, x2_scr, y2_scr, ar_scr):
    b = pl.program_id(0)
    nb = pl.num_programs(0)

    blk = x_ref[0]
    cx = blk[0:1, :]
    cy = blk[1:2, :]
    w = blk[2:3, :]
    h = blk[3:4, :]
    x1 = cx - w / 2
    y1 = cy - h / 2
    x2 = cx + w / 2
    y2 = cy + h / 2

    scores = blk[4:, :]
    m = jnp.max(scores, axis=0, keepdims=True)
    iota_c = lax.broadcasted_iota(jnp.int32, scores.shape, 0)
    cls = jnp.min(jnp.where(scores == m, iota_c, _BIG_I32),
                  axis=0, keepdims=True)

    s_scr[pl.ds(b, 1), :] = jnp.where(m > _SCORE_THR, m, -1.0)
    cls_scr[pl.ds(b, 1), :] = cls
    x1_scr[pl.ds(b, 1), :] = x1
    y1_scr[pl.ds(b, 1), :] = y1
    x2_scr[pl.ds(b, 1), :] = x2
    y2_scr[pl.ds(b, 1), :] = y2
    ar_scr[pl.ds(b, 1), :] = (x2 - x1) * (y2 - y1)

    @pl.when(b == nb - 1)
    def _phase2():
        S0 = s_scr[...]
        X1 = x1_scr[...]
        Y1 = y1_scr[...]
        X2 = x2_scr[...]
        Y2 = y2_scr[...]
        AREA = ar_scr[...]
        CLS = cls_scr[...]
        B, N = S0.shape
        iota_n = lax.broadcasted_iota(jnp.int32, (B, N), 1)
        col = lax.broadcasted_iota(jnp.int32, (B, _MAX_OBJ), 1)

        def body(t, carry):
            S, nd, asc, acl, aix, ab0, ab1, ab2, ab3 = carry
            best = jnp.max(S, axis=1, keepdims=True)
            idx = jnp.min(jnp.where(S == best, iota_n, _BIG_I32),
                          axis=1, keepdims=True)
            onehot = iota_n == idx
            bx1 = jnp.sum(jnp.where(onehot, X1, 0.0), axis=1, keepdims=True)
            by1 = jnp.sum(jnp.where(onehot, Y1, 0.0), axis=1, keepdims=True)
            bx2 = jnp.sum(jnp.where(onehot, X2, 0.0), axis=1, keepdims=True)
            by2 = jnp.sum(jnp.where(onehot, Y2, 0.0), axis=1, keepdims=True)
            bcl = jnp.sum(jnp.where(onehot, CLS, 0), axis=1, keepdims=True)

            ix1 = jnp.maximum(bx1, X1)
            iy1 = jnp.maximum(by1, Y1)
            ix2 = jnp.minimum(bx2, X2)
            iy2 = jnp.minimum(by2, Y2)
            inter = jnp.clip(ix2 - ix1, 0.0) * jnp.clip(iy2 - iy1, 0.0)
            area1 = (bx2 - bx1) * (by2 - by1)
            iou = inter / (area1 + AREA - inter + 1e-9)

            S = jnp.where(iou > _IOU_THR, -1.0, S)
            S = jnp.where(onehot, -1.0, S)

            keep = best > _SCORE_THR
            sel = col == t
            asc = jnp.where(sel, jnp.where(keep, best, 0.0), asc)
            acl = jnp.where(sel, jnp.where(keep, bcl, -1), acl)
            aix = jnp.where(sel, idx, aix)
            ab0 = jnp.where(sel, jnp.where(keep, (bx1 + bx2) * 0.5, 0.0), ab0)
            ab1 = jnp.where(sel, jnp.where(keep, (by1 + by2) * 0.5, 0.0), ab1)
            ab2 = jnp.where(sel, jnp.where(keep, bx2 - bx1, 0.0), ab2)
            ab3 = jnp.where(sel, jnp.where(keep, by2 - by1, 0.0), ab3)
            nd = nd + keep.astype(jnp.int32)
            return (S, nd, asc, acl, aix, ab0, ab1, ab2, ab3)

        init = (S0,
                jnp.zeros((B, 1), jnp.int32),
                jnp.zeros((B, _MAX_OBJ), jnp.float32),
                jnp.zeros((B, _MAX_OBJ), jnp.int32),
                jnp.zeros((B, _MAX_OBJ), jnp.int32),
                jnp.zeros((B, _MAX_OBJ), jnp.float32),
                jnp.zeros((B, _MAX_OBJ), jnp.float32),
                jnp.zeros((B, _MAX_OBJ), jnp.float32),
                jnp.zeros((B, _MAX_OBJ), jnp.float32))
        (_, nd, asc, acl, aix, ab0, ab1, ab2, ab3) = lax.fori_loop(
            0, _MAX_OBJ, body, init)

        nd_ref[...] = nd
        sc_ref[...] = asc
        cl_ref[...] = acl
        ix_ref[...] = aix
        b0_ref[...] = ab0
        b1_ref[...] = ab1
        b2_ref[...] = ab2
        b3_ref[...] = ab3


def kernel(x):
    B, C, N = x.shape
    out_shapes = (
        jax.ShapeDtypeStruct((B, 1), jnp.int32),
        jax.ShapeDtypeStruct((B, _MAX_OBJ), jnp.float32),
        jax.ShapeDtypeStruct((B, _MAX_OBJ), jnp.int32),
        jax.ShapeDtypeStruct((B, _MAX_OBJ), jnp.int32),
        jax.ShapeDtypeStruct((B, _MAX_OBJ), jnp.float32),
        jax.ShapeDtypeStruct((B, _MAX_OBJ), jnp.float32),
        jax.ShapeDtypeStruct((B, _MAX_OBJ), jnp.float32),
        jax.ShapeDtypeStruct((B, _MAX_OBJ), jnp.float32),
    )
    out_specs = tuple(
        pl.BlockSpec(s.shape, lambda b: (0,) * 2) for s in out_shapes)
    scratch = [
        pltpu.VMEM((B, N), jnp.float32),
        pltpu.VMEM((B, N), jnp.int32),
        pltpu.VMEM((B, N), jnp.float32),
        pltpu.VMEM((B, N), jnp.float32),
        pltpu.VMEM((B, N), jnp.float32),
        pltpu.VMEM((B, N), jnp.float32),
        pltpu.VMEM((B, N), jnp.float32),
    ]
    nd, asc, acl, aix, ab0, ab1, ab2, ab3 = pl.pallas_call(
        _nms_tc_kernel,
        grid=(B,),
        in_specs=[pl.BlockSpec((1, C, N), lambda b: (b, 0, 0))],
        out_specs=out_specs,
        out_shape=out_shapes,
        scratch_shapes=scratch,
    )(x)
    det_boxes = jnp.stack([ab0, ab1, ab2, ab3], axis=-1)
    return (nd, det_boxes, asc, acl, aix)
